# baseline (speedup 1.0000x reference)
import functools

import jax
import jax.numpy as jnp
from jax import lax
from jax.experimental import pallas as pl
from jax.experimental.pallas import tpu as pltpu
from jax.experimental.pallas import tpu_sc as plsc

FDIM = 128

_NC = 2
_NS = 16
_NW = _NC * _NS

_SC_PARAMS = pltpu.CompilerParams(use_tc_tiling_on_sc=False,
                                  needs_layout_passes=False)
_MESH = dict(core_axis_name="c", subcore_axis_name="s")


def _spmm_body(nch, rpt, chunk, eper, z_hbm, srcp, dstp, out_hbm,
               src_v, dst_v, b0, b1, b2, b3, acc,
               g0, g1, g2, g3, s0, s1, s2, s3):
    c = lax.axis_index("c")
    s = lax.axis_index("s")
    wid = c * _NS + s
    bufs = (b0, b1, b2, b3)
    gsem = (g0, g1, g2, g3)
    ssem = (s0, s1, s2, s3)

    pltpu.sync_copy(srcp.at[pl.ds(wid * eper, eper)], src_v)
    pltpu.sync_copy(dstp.at[pl.ds(wid * eper, eper)], dst_v)

    base = s * rpt
    pltpu.sync_copy(z_hbm.at[pl.ds(base, rpt)], acc.at[pl.ds(base, rpt)])
    plsc.subcore_barrier()

    def idx(ref, j):
        return ref.at[pl.ds(j * chunk, chunk)]

    def gather(j, k):
        return pltpu.async_copy(z_hbm.at[idx(src_v, j)], bufs[k], gsem[k])

    def gather_wait(j, k):
        pltpu.make_async_copy(z_hbm.at[idx(src_v, j)], bufs[k],
                              gsem[k]).wait()

    def scatter(j, k):
        return pltpu.async_copy(bufs[k], acc.at[idx(dst_v, j)], ssem[k],
                                add=True)

    def scatter_wait(j, k):
        pltpu.make_async_copy(bufs[k], acc.at[idx(dst_v, j)], ssem[k]).wait()

    nb = len(bufs)
    for j in range(nb - 1):
        gather(j, j)
    gather_wait(0, 0)
    scatter(0, 0)
    gather(nb - 1, nb - 1)

    def body(i, carry):
        for dj in range(nb):
            j = nb * i + 1 + dj
            k = (1 + dj) % nb
            kn = dj % nb
            gather_wait(j, k)
            scatter(j, k)
            scatter_wait(j - 1, kn)
            gather(j + nb - 1, kn)
        return carry

    lax.fori_loop(0, (nch - nb) // nb, body, 0)
    for dj in range(nb - 1):
        j = nch - (nb - 1) + dj
        gather_wait(j, j % nb)
        scatter(j, j % nb)
    for j in range(nch - nb, nch):
        scatter_wait(j, j % nb)
    plsc.subcore_barrier()

    pltpu.sync_copy(acc.at[pl.ds(base, rpt)], out_hbm.at[c, pl.ds(base, rpt)])


@functools.lru_cache(maxsize=None)
def _make_spmm(npad, eper, chunk):
    rpt = npad // _NS
    nch = eper // chunk
    sem = pltpu.SemaphoreType.DMA
    return pl.kernel(
        functools.partial(_spmm_body, nch, rpt, chunk, eper),
        out_type=jax.ShapeDtypeStruct((_NC, npad, FDIM), jnp.float32),
        mesh=plsc.VectorSubcoreMesh(**_MESH),
        compiler_params=_SC_PARAMS,
        scratch_types=[
            pltpu.VMEM((eper,), jnp.int32),
            pltpu.VMEM((eper,), jnp.int32),
            pltpu.VMEM((chunk, FDIM), jnp.float32),
            pltpu.VMEM((chunk, FDIM), jnp.float32),
            pltpu.VMEM((chunk, FDIM), jnp.float32),
            pltpu.VMEM((chunk, FDIM), jnp.float32),
            pltpu.VMEM_SHARED((npad, FDIM), jnp.float32),
            sem, sem, sem, sem, sem, sem, sem, sem,
        ],
    )


def _publish_reduce(s, c, part_v, shared, tmp, red, out_hbm, seg):
    pltpu.sync_copy(part_v, shared.at[s])
    plsc.subcore_barrier()
    sl = pl.ds(s * seg, seg)
    pltpu.sync_copy(shared.at[0, sl], red)

    def rw(w, carry):
        pltpu.sync_copy(shared.at[w, sl], tmp)

        def rr(i, c2):
            v = pl.ds(i * 16, 16)
            red[v] = red[v] + tmp[v]
            return c2

        lax.fori_loop(0, seg // 16, rr, 0)
        return carry

    lax.fori_loop(1, _NS, rw, 0)
    pltpu.sync_copy(red, out_hbm.at[c, pl.ds(s * seg, seg)])


def _deg_body(eper, npad, dstp, out_hbm, dst_v, hist, tmp, red, shared):
    c = lax.axis_index("c")
    s = lax.axis_index("s")
    wid = c * _NS + s
    pltpu.sync_copy(dstp.at[pl.ds(wid * eper, eper)], dst_v)

    def z(i, carry):
        hist[pl.ds(i * 16, 16)] = jnp.zeros((16,), jnp.float32)
        return carry

    lax.fori_loop(0, npad // 16, z, 0)
    ones = jnp.ones((16,), jnp.float32)

    def b(i, carry):
        idx = dst_v[pl.ds(i * 16, 16)]
        plsc.addupdate_scatter(hist, [idx], ones)
        return carry

    lax.fori_loop(0, eper // 16, b, 0)
    _publish_reduce(s, c, hist, shared, tmp, red, out_hbm, npad // _NS)


@functools.lru_cache(maxsize=None)
def _make_deg(npad, eper):
    seg = npad // _NS
    return pl.kernel(
        functools.partial(_deg_body, eper, npad),
        out_type=jax.ShapeDtypeStruct((_NC, npad), jnp.float32),
        mesh=plsc.VectorSubcoreMesh(**_MESH),
        compiler_params=_SC_PARAMS,
        scratch_types=[
            pltpu.VMEM((eper,), jnp.int32),
            pltpu.VMEM((npad,), jnp.float32),
            pltpu.VMEM((seg,), jnp.float32),
            pltpu.VMEM((seg,), jnp.float32),
            pltpu.VMEM_SHARED((_NS, npad), jnp.float32),
        ],
    )


def _l2_body(eper, npad, zg_hbm, srcp, dstp, out_hbm,
             src_v, dst_v, zg_v, hist, tmp, red, shared):
    c = lax.axis_index("c")
    s = lax.axis_index("s")
    wid = c * _NS + s
    pltpu.sync_copy(srcp.at[pl.ds(wid * eper, eper)], src_v)
    pltpu.sync_copy(dstp.at[pl.ds(wid * eper, eper)], dst_v)
    pltpu.sync_copy(zg_hbm, zg_v)

    def z(i, carry):
        hist[pl.ds(i * 16, 16)] = jnp.zeros((16,), jnp.float32)
        return carry

    lax.fori_loop(0, (npad * 2) // 16, z, 0)

    def b(i, carry):
        si = src_v[pl.ds(i * 16, 16)] * 2
        di = dst_v[pl.ds(i * 16, 16)] * 2
        v0 = plsc.load_gather(zg_v, [si])
        v1 = plsc.load_gather(zg_v, [si + 1])
        plsc.addupdate_scatter(hist, [di], v0)
        plsc.addupdate_scatter(hist, [di + 1], v1)
        return carry

    lax.fori_loop(0, eper // 16, b, 0)
    _publish_reduce(s, c, hist, shared, tmp, red, out_hbm,
                    (npad * 2) // _NS)


@functools.lru_cache(maxsize=None)
def _make_l2(npad, eper):
    seg = (npad * 2) // _NS
    return pl.kernel(
        functools.partial(_l2_body, eper, npad),
        out_type=jax.ShapeDtypeStruct((_NC, npad * 2), jnp.float32),
        mesh=plsc.VectorSubcoreMesh(**_MESH),
        compiler_params=_SC_PARAMS,
        scratch_types=[
            pltpu.VMEM((eper,), jnp.int32),
            pltpu.VMEM((eper,), jnp.int32),
            pltpu.VMEM((npad * 2,), jnp.float32),
            pltpu.VMEM((npad * 2,), jnp.float32),
            pltpu.VMEM((seg,), jnp.float32),
            pltpu.VMEM((seg,), jnp.float32),
            pltpu.VMEM_SHARED((_NS, npad * 2), jnp.float32),
        ],
    )


def _mm_body(x_ref, w_ref, o_ref):
    o_ref[...] = jnp.dot(x_ref[...], w_ref[...],
                         preferred_element_type=jnp.float32)


def _matmul(x, w, blk):
    m, k = x.shape
    _, n = w.shape
    return pl.pallas_call(
        _mm_body,
        grid=(m // blk,),
        in_specs=[
            pl.BlockSpec((blk, k), lambda i: (i, 0)),
            pl.BlockSpec((k, n), lambda i: (0, 0)),
        ],
        out_specs=pl.BlockSpec((blk, n), lambda i: (i, 0)),
        out_shape=jax.ShapeDtypeStruct((m, n), jnp.float32),
    )(x, w)


def _scale_body(d_ref, y_ref, o_ref):
    o_ref[...] = lax.rsqrt(d_ref[...]) * y_ref[...]


def _scale(degc, y, blk):
    m, k = y.shape
    return pl.pallas_call(
        _scale_body,
        grid=(m // blk,),
        in_specs=[
            pl.BlockSpec((blk, 1), lambda i: (i, 0)),
            pl.BlockSpec((blk, k), lambda i: (i, 0)),
        ],
        out_specs=pl.BlockSpec((blk, k), lambda i: (i, 0)),
        out_shape=jax.ShapeDtypeStruct((m, k), jnp.float32),
    )(degc, y)


def _combine_mm_body(d_ref, a_ref, z_ref, b_ref, w_ref, o_ref):
    dis = lax.rsqrt(d_ref[...])
    h = dis * (a_ref[0] + a_ref[1] - z_ref[...]) + b_ref[...]
    h = jnp.maximum(h, 0.0)
    o_ref[...] = dis * jnp.dot(h, w_ref[...],
                               preferred_element_type=jnp.float32)


def _combine_matmul(degc, acc, z, b1, w2, blk):
    m, k = z.shape
    n = w2.shape[1]
    return pl.pallas_call(
        _combine_mm_body,
        grid=(m // blk,),
        in_specs=[
            pl.BlockSpec((blk, 1), lambda i: (i, 0)),
            pl.BlockSpec((2, blk, k), lambda i: (0, i, 0)),
            pl.BlockSpec((blk, k), lambda i: (i, 0)),
            pl.BlockSpec((1, k), lambda i: (0, 0)),
            pl.BlockSpec((k, n), lambda i: (0, 0)),
        ],
        out_specs=pl.BlockSpec((blk, n), lambda i: (i, 0)),
        out_shape=jax.ShapeDtypeStruct((m, n), jnp.float32),
    )(degc, acc, z, b1, w2)


def _epilogue_body(d_ref, a2_ref, zg_ref, b2_ref, o_ref):
    dis = lax.rsqrt(d_ref[...])
    o_ref[...] = dis * (a2_ref[0] + a2_ref[1] + zg_ref[...]) + b2_ref[...]


def _epilogue(degc, acc2, zg2, b2r, n):
    return pl.pallas_call(
        _epilogue_body,
        grid=(1,),
        in_specs=[
            pl.BlockSpec((n, 1), lambda i: (0, 0)),
            pl.BlockSpec((2, n, 2), lambda i: (0, 0, 0)),
            pl.BlockSpec((n, 2), lambda i: (0, 0)),
            pl.BlockSpec((1, 2), lambda i: (0, 0)),
        ],
        out_specs=pl.BlockSpec((n, 2), lambda i: (0, 0)),
        out_shape=jax.ShapeDtypeStruct((n, 2), jnp.float32),
    )(degc, acc2, zg2, b2r)


def kernel(x, edge_index, W1, b1, W2, b2):
    n = x.shape[0]
    e = edge_index.shape[1]
    npad = ((n + 2047) // 2048) * 2048
    blk = npad // 5
    ei = edge_index.astype(jnp.int32)

    grain = _NW * 128 * 3
    epad = ((e + grain - 1) // grain) * grain
    eper = epad // _NW
    fill = jnp.full((epad - e,), n, jnp.int32)
    srcp = jnp.concatenate([ei[0], fill])
    dstp = jnp.concatenate([ei[1], fill])

    dacc = _make_deg(npad, eper)(dstp)
    xp = jnp.pad(x, ((0, npad - n), (0, 0)))
    y = _matmul(xp, W1, blk)
    degc = (dacc[0] + dacc[1] + 1.0)[:, None]

    z = _scale(degc, y, blk)
    acc = _make_spmm(npad, eper, 48)(z, srcp, dstp)
    zg2 = _combine_matmul(degc, acc, z, b1[None, :], W2, blk)

    s2 = _make_l2(npad, eper)(zg2.reshape(-1), srcp, dstp)
    return _epilogue(degc, s2.reshape(_NC, npad, 2), zg2,
                     b2[None, :], n)


# speedup vs baseline: 3.4909x; 3.4909x over previous
import functools

import jax
import jax.numpy as jnp
from jax import lax
from jax.experimental import pallas as pl
from jax.experimental.pallas import tpu as pltpu
from jax.experimental.pallas import tpu_sc as plsc

FDIM = 128

_NC = 2
_NS = 16
_NW = _NC * _NS

_SC_PARAMS = pltpu.CompilerParams(use_tc_tiling_on_sc=False,
                                  needs_layout_passes=False)
_MESH = dict(core_axis_name="c", subcore_axis_name="s")


def _spmm_body(nch, rpt, chunk, eper, z_hbm, srcp, dstp, out_hbm,
               src_v, dst_v, b0, b1, b2, b3, acc,
               g0, g1, g2, g3, s0, s1, s2, s3):
    c = lax.axis_index("c")
    s = lax.axis_index("s")
    wid = c * _NS + s
    bufs = (b0, b1, b2, b3)
    gsem = (g0, g1, g2, g3)
    ssem = (s0, s1, s2, s3)

    pltpu.sync_copy(srcp.at[pl.ds(wid * eper, eper)], src_v)
    pltpu.sync_copy(dstp.at[pl.ds(wid * eper, eper)], dst_v)

    base = s * rpt
    pltpu.sync_copy(z_hbm.at[pl.ds(base, rpt)], acc.at[pl.ds(base, rpt)])
    plsc.subcore_barrier()

    def idx(ref, j):
        return ref.at[pl.ds(j * chunk, chunk)]

    def gather(j, k):
        return pltpu.async_copy(z_hbm.at[idx(src_v, j)], bufs[k], gsem[k])

    def gather_wait(j, k):
        pltpu.make_async_copy(z_hbm.at[idx(src_v, j)], bufs[k],
                              gsem[k]).wait()

    def scatter(j, k):
        return pltpu.async_copy(bufs[k], acc.at[idx(dst_v, j)], ssem[k],
                                add=True)

    def scatter_wait(j, k):
        pltpu.make_async_copy(bufs[k], acc.at[idx(dst_v, j)], ssem[k]).wait()

    nb = len(bufs)
    for j in range(nb - 1):
        gather(j, j)
    gather_wait(0, 0)
    scatter(0, 0)
    gather(nb - 1, nb - 1)

    def body(i, carry):
        for dj in range(nb):
            j = nb * i + 1 + dj
            k = (1 + dj) % nb
            kn = dj % nb
            gather_wait(j, k)
            scatter(j, k)
            scatter_wait(j - 1, kn)
            gather(j + nb - 1, kn)
        return carry

    lax.fori_loop(0, (nch - nb) // nb, body, 0)
    for dj in range(nb - 1):
        j = nch - (nb - 1) + dj
        gather_wait(j, j % nb)
        scatter(j, j % nb)
    for j in range(nch - nb, nch):
        scatter_wait(j, j % nb)
    plsc.subcore_barrier()

    pltpu.sync_copy(acc.at[pl.ds(base, rpt)], out_hbm.at[c, pl.ds(base, rpt)])


@functools.lru_cache(maxsize=None)
def _make_spmm(npad, eper, chunk):
    rpt = npad // _NS
    nch = eper // chunk
    sem = pltpu.SemaphoreType.DMA
    return pl.kernel(
        functools.partial(_spmm_body, nch, rpt, chunk, eper),
        out_type=jax.ShapeDtypeStruct((_NC, npad, FDIM), jnp.float32),
        mesh=plsc.VectorSubcoreMesh(**_MESH),
        compiler_params=_SC_PARAMS,
        scratch_types=[
            pltpu.VMEM((eper,), jnp.int32),
            pltpu.VMEM((eper,), jnp.int32),
            pltpu.VMEM((chunk, FDIM), jnp.float32),
            pltpu.VMEM((chunk, FDIM), jnp.float32),
            pltpu.VMEM((chunk, FDIM), jnp.float32),
            pltpu.VMEM((chunk, FDIM), jnp.float32),
            pltpu.VMEM_SHARED((npad, FDIM), jnp.float32),
            sem, sem, sem, sem, sem, sem, sem, sem,
        ],
    )


def _publish_reduce(s, c, part_v, shared, tmp, red, out_hbm, seg):
    pltpu.sync_copy(part_v, shared.at[s])
    plsc.subcore_barrier()
    sl = pl.ds(s * seg, seg)
    pltpu.sync_copy(shared.at[0, sl], red)

    def rw(w, carry):
        pltpu.sync_copy(shared.at[w, sl], tmp)

        def rr(i, c2):
            v = pl.ds(i * 16, 16)
            red[v] = red[v] + tmp[v]
            return c2

        lax.fori_loop(0, seg // 16, rr, 0)
        return carry

    lax.fori_loop(1, _NS, rw, 0)
    pltpu.sync_copy(red, out_hbm.at[c, pl.ds(s * seg, seg)])


def _deg_body(eper, npad, dstp, out_hbm, dst_v, hist, tmp, red, shared):
    c = lax.axis_index("c")
    s = lax.axis_index("s")
    wid = c * _NS + s
    pltpu.sync_copy(dstp.at[pl.ds(wid * eper, eper)], dst_v)

    def z(i, carry):
        hist[pl.ds(i * 16, 16)] = jnp.zeros((16,), jnp.float32)
        return carry

    lax.fori_loop(0, npad // 16, z, 0)
    ones = jnp.ones((16,), jnp.float32)

    def b(i, carry):
        idx = dst_v[pl.ds(i * 16, 16)]
        plsc.addupdate_scatter(hist, [idx], ones)
        return carry

    lax.fori_loop(0, eper // 16, b, 0)
    _publish_reduce(s, c, hist, shared, tmp, red, out_hbm, npad // _NS)


@functools.lru_cache(maxsize=None)
def _make_deg(npad, eper):
    seg = npad // _NS
    return pl.kernel(
        functools.partial(_deg_body, eper, npad),
        out_type=jax.ShapeDtypeStruct((_NC, npad), jnp.float32),
        mesh=plsc.VectorSubcoreMesh(**_MESH),
        compiler_params=_SC_PARAMS,
        scratch_types=[
            pltpu.VMEM((eper,), jnp.int32),
            pltpu.VMEM((npad,), jnp.float32),
            pltpu.VMEM((seg,), jnp.float32),
            pltpu.VMEM((seg,), jnp.float32),
            pltpu.VMEM_SHARED((_NS, npad), jnp.float32),
        ],
    )


def _l2_body(eper, npad, zg_hbm, srcp, dstp, out_hbm,
             src_v, dst_v, zg_v, hist, tmp, red, shared):
    c = lax.axis_index("c")
    s = lax.axis_index("s")
    wid = c * _NS + s
    pltpu.sync_copy(srcp.at[pl.ds(wid * eper, eper)], src_v)
    pltpu.sync_copy(dstp.at[pl.ds(wid * eper, eper)], dst_v)
    pltpu.sync_copy(zg_hbm, zg_v)

    def z(i, carry):
        hist[pl.ds(i * 16, 16)] = jnp.zeros((16,), jnp.float32)
        return carry

    lax.fori_loop(0, (npad * 2) // 16, z, 0)

    def b(i, carry):
        si = src_v[pl.ds(i * 16, 16)] * 2
        di = dst_v[pl.ds(i * 16, 16)] * 2
        v0 = plsc.load_gather(zg_v, [si])
        v1 = plsc.load_gather(zg_v, [si + 1])
        plsc.addupdate_scatter(hist, [di], v0)
        plsc.addupdate_scatter(hist, [di + 1], v1)
        return carry

    lax.fori_loop(0, eper // 16, b, 0)
    _publish_reduce(s, c, hist, shared, tmp, red, out_hbm,
                    (npad * 2) // _NS)


@functools.lru_cache(maxsize=None)
def _make_l2(npad, eper):
    seg = (npad * 2) // _NS
    return pl.kernel(
        functools.partial(_l2_body, eper, npad),
        out_type=jax.ShapeDtypeStruct((_NC, npad * 2), jnp.float32),
        mesh=plsc.VectorSubcoreMesh(**_MESH),
        compiler_params=_SC_PARAMS,
        scratch_types=[
            pltpu.VMEM((eper,), jnp.int32),
            pltpu.VMEM((eper,), jnp.int32),
            pltpu.VMEM((npad * 2,), jnp.float32),
            pltpu.VMEM((npad * 2,), jnp.float32),
            pltpu.VMEM((seg,), jnp.float32),
            pltpu.VMEM((seg,), jnp.float32),
            pltpu.VMEM_SHARED((_NS, npad * 2), jnp.float32),
        ],
    )


def _mm_body(x_ref, w_ref, o_ref):
    o_ref[...] = jnp.dot(x_ref[...], w_ref[...],
                         preferred_element_type=jnp.float32)


def _matmul(x, w, blk):
    m, k = x.shape
    _, n = w.shape
    return pl.pallas_call(
        _mm_body,
        grid=(m // blk,),
        in_specs=[
            pl.BlockSpec((blk, k), lambda i: (i, 0)),
            pl.BlockSpec((k, n), lambda i: (0, 0)),
        ],
        out_specs=pl.BlockSpec((blk, n), lambda i: (i, 0)),
        out_shape=jax.ShapeDtypeStruct((m, n), jnp.float32),
    )(x, w)


def _scale_body(d_ref, y_ref, o_ref):
    o_ref[...] = lax.rsqrt(d_ref[...]) * y_ref[...]


def _scale(degc, y, blk):
    m, k = y.shape
    return pl.pallas_call(
        _scale_body,
        grid=(m // blk,),
        in_specs=[
            pl.BlockSpec((blk, 1), lambda i: (i, 0)),
            pl.BlockSpec((blk, k), lambda i: (i, 0)),
        ],
        out_specs=pl.BlockSpec((blk, k), lambda i: (i, 0)),
        out_shape=jax.ShapeDtypeStruct((m, k), jnp.float32),
    )(degc, y)


def _combine_mm_body(d_ref, a_ref, z_ref, b_ref, w_ref, o_ref):
    dis = lax.rsqrt(d_ref[...])
    h = dis * (a_ref[0] + a_ref[1] - z_ref[...]) + b_ref[...]
    h = jnp.maximum(h, 0.0)
    o_ref[...] = dis * jnp.dot(h, w_ref[...],
                               preferred_element_type=jnp.float32)


def _combine_matmul(degc, acc, z, b1, w2, blk):
    m, k = z.shape
    n = w2.shape[1]
    return pl.pallas_call(
        _combine_mm_body,
        grid=(m // blk,),
        in_specs=[
            pl.BlockSpec((blk, 1), lambda i: (i, 0)),
            pl.BlockSpec((2, blk, k), lambda i: (0, i, 0)),
            pl.BlockSpec((blk, k), lambda i: (i, 0)),
            pl.BlockSpec((1, k), lambda i: (0, 0)),
            pl.BlockSpec((k, n), lambda i: (0, 0)),
        ],
        out_specs=pl.BlockSpec((blk, n), lambda i: (i, 0)),
        out_shape=jax.ShapeDtypeStruct((m, n), jnp.float32),
    )(degc, acc, z, b1, w2)


def _epilogue_body(d_ref, a2_ref, zg_ref, b2_ref, o_ref):
    dis = lax.rsqrt(d_ref[...])
    o_ref[...] = dis * (a2_ref[0] + a2_ref[1] + zg_ref[...]) + b2_ref[...]


def _epilogue(degc, acc2, zg2, b2r, n):
    return pl.pallas_call(
        _epilogue_body,
        grid=(1,),
        in_specs=[
            pl.BlockSpec((n, 1), lambda i: (0, 0)),
            pl.BlockSpec((2, n, 2), lambda i: (0, 0, 0)),
            pl.BlockSpec((n, 2), lambda i: (0, 0)),
            pl.BlockSpec((1, 2), lambda i: (0, 0)),
        ],
        out_specs=pl.BlockSpec((n, 2), lambda i: (0, 0)),
        out_shape=jax.ShapeDtypeStruct((n, 2), jnp.float32),
    )(degc, acc2, zg2, b2r)


def kernel(x, edge_index, W1, b1, W2, b2):
    n = x.shape[0]
    e = edge_index.shape[1]
    npad = ((n + 2047) // 2048) * 2048
    blk = npad // 5
    ei = edge_index.astype(jnp.int32)

    grain = _NW * 128 * 3
    epad = ((e + grain - 1) // grain) * grain
    eper = epad // _NW
    fill = n + (jnp.arange(epad - e, dtype=jnp.int32) % (npad - n))
    srcp = jnp.concatenate([ei[0], fill])
    dstp = jnp.concatenate([ei[1], fill])

    dacc = _make_deg(npad, eper)(dstp)
    xp = jnp.pad(x, ((0, npad - n), (0, 0)))
    y = _matmul(xp, W1, blk)
    degc = (dacc[0] + dacc[1] + 1.0)[:, None]

    z = _scale(degc, y, blk)
    acc = _make_spmm(npad, eper, 48)(z, srcp, dstp)
    zg2 = _combine_matmul(degc, acc, z, b1[None, :], W2, blk)

    s2 = _make_l2(npad, eper)(zg2.reshape(-1), srcp, dstp)
    return _epilogue(degc, s2.reshape(_NC, npad, 2), zg2,
                     b2[None, :], n)


# speedup vs baseline: 3.7368x; 1.0704x over previous
import functools

import jax
import jax.numpy as jnp
from jax import lax
from jax.experimental import pallas as pl
from jax.experimental.pallas import tpu as pltpu
from jax.experimental.pallas import tpu_sc as plsc

FDIM = 128

_NC = 2
_NS = 16
_NW = _NC * _NS

_SC_PARAMS = pltpu.CompilerParams(use_tc_tiling_on_sc=False,
                                  needs_layout_passes=False)
_MESH = dict(core_axis_name="c", subcore_axis_name="s")


def _spmm_body(nch, rpt, chunk, eper, z_hbm, srcp, dstp, out_hbm,
               src_v, dst_v, b0, b1, b2, b3, acc,
               g0, g1, g2, g3, s0, s1, s2, s3):
    c = lax.axis_index("c")
    s = lax.axis_index("s")
    wid = c * _NS + s
    bufs = (b0, b1, b2, b3)
    gsem = (g0, g1, g2, g3)
    ssem = (s0, s1, s2, s3)

    pltpu.sync_copy(srcp.at[pl.ds(wid * eper, eper)], src_v)
    pltpu.sync_copy(dstp.at[pl.ds(wid * eper, eper)], dst_v)

    base = s * rpt
    pltpu.sync_copy(z_hbm.at[pl.ds(base, rpt)], acc.at[pl.ds(base, rpt)])
    plsc.subcore_barrier()

    def idx(ref, j):
        return ref.at[pl.ds(j * chunk, chunk)]

    def gather(j, k):
        return pltpu.async_copy(z_hbm.at[idx(src_v, j)], bufs[k], gsem[k])

    def gather_wait(j, k):
        pltpu.make_async_copy(z_hbm.at[idx(src_v, j)], bufs[k],
                              gsem[k]).wait()

    def scatter(j, k):
        return pltpu.async_copy(bufs[k], acc.at[idx(dst_v, j)], ssem[k],
                                add=True)

    def scatter_wait(j, k):
        pltpu.make_async_copy(bufs[k], acc.at[idx(dst_v, j)], ssem[k]).wait()

    nb = len(bufs)
    for j in range(nb - 1):
        gather(j, j)
    gather_wait(0, 0)
    scatter(0, 0)
    gather(nb - 1, nb - 1)

    def body(i, carry):
        for dj in range(nb):
            j = nb * i + 1 + dj
            k = (1 + dj) % nb
            kn = dj % nb
            gather_wait(j, k)
            scatter(j, k)
            scatter_wait(j - 1, kn)
            gather(j + nb - 1, kn)
        return carry

    lax.fori_loop(0, (nch - nb) // nb, body, 0)
    for dj in range(nb - 1):
        j = nch - (nb - 1) + dj
        gather_wait(j, j % nb)
        scatter(j, j % nb)
    for j in range(nch - nb, nch):
        scatter_wait(j, j % nb)
    plsc.subcore_barrier()

    pltpu.sync_copy(acc.at[pl.ds(base, rpt)], out_hbm.at[c, pl.ds(base, rpt)])


@functools.lru_cache(maxsize=None)
def _make_spmm(npad, eper, chunk):
    rpt = npad // _NS
    nch = eper // chunk
    sem = pltpu.SemaphoreType.DMA
    return pl.kernel(
        functools.partial(_spmm_body, nch, rpt, chunk, eper),
        out_type=jax.ShapeDtypeStruct((_NC, npad, FDIM), jnp.float32),
        mesh=plsc.VectorSubcoreMesh(**_MESH),
        compiler_params=_SC_PARAMS,
        scratch_types=[
            pltpu.VMEM((eper,), jnp.int32),
            pltpu.VMEM((eper,), jnp.int32),
            pltpu.VMEM((chunk, FDIM), jnp.float32),
            pltpu.VMEM((chunk, FDIM), jnp.float32),
            pltpu.VMEM((chunk, FDIM), jnp.float32),
            pltpu.VMEM((chunk, FDIM), jnp.float32),
            pltpu.VMEM_SHARED((npad, FDIM), jnp.float32),
            sem, sem, sem, sem, sem, sem, sem, sem,
        ],
    )


def _publish_reduce(s, c, part_v, shared, tmp, red, out_hbm, seg):
    pltpu.sync_copy(part_v, shared.at[s])
    plsc.subcore_barrier()
    pltpu.sync_copy(shared.at[:, pl.ds(s * seg, seg)], tmp)

    def rr(i, c2):
        v = pl.ds(i * 16, 16)
        a = tmp[0, v]
        for w in range(1, _NS):
            a = a + tmp[w, v]
        red[v] = a
        return c2

    lax.fori_loop(0, seg // 16, rr, 0)
    pltpu.sync_copy(red, out_hbm.at[c, pl.ds(s * seg, seg)])


def _deg_body(eper, npad, dstp, out_hbm, dst_v, hist, tmp, red, shared):
    c = lax.axis_index("c")
    s = lax.axis_index("s")
    wid = c * _NS + s
    pltpu.sync_copy(dstp.at[pl.ds(wid * eper, eper)], dst_v)

    def z(i, carry):
        hist[pl.ds(i * 16, 16)] = jnp.zeros((16,), jnp.float32)
        return carry

    lax.fori_loop(0, npad // 16, z, 0)
    ones = jnp.ones((16,), jnp.float32)

    def b(i, carry):
        idx = dst_v[pl.ds(i * 16, 16)]
        plsc.addupdate_scatter(hist, [idx], ones)
        return carry

    lax.fori_loop(0, eper // 16, b, 0)
    _publish_reduce(s, c, hist, shared, tmp, red, out_hbm, npad // _NS)


@functools.lru_cache(maxsize=None)
def _make_deg(npad, eper):
    seg = npad // _NS
    return pl.kernel(
        functools.partial(_deg_body, eper, npad),
        out_type=jax.ShapeDtypeStruct((_NC, npad), jnp.float32),
        mesh=plsc.VectorSubcoreMesh(**_MESH),
        compiler_params=_SC_PARAMS,
        scratch_types=[
            pltpu.VMEM((eper,), jnp.int32),
            pltpu.VMEM((npad,), jnp.float32),
            pltpu.VMEM((_NS, seg), jnp.float32),
            pltpu.VMEM((seg,), jnp.float32),
            pltpu.VMEM_SHARED((_NS, npad), jnp.float32),
        ],
    )


def _l2_body(eper, npad, zg_hbm, srcp, dstp, out_hbm,
             src_v, dst_v, zg_v, hist, tmp, red, shared, m0, m1, m2):
    c = lax.axis_index("c")
    s = lax.axis_index("s")
    wid = c * _NS + s
    d0 = pltpu.async_copy(srcp.at[pl.ds(wid * eper, eper)], src_v, m0)
    d1 = pltpu.async_copy(dstp.at[pl.ds(wid * eper, eper)], dst_v, m1)
    d2 = pltpu.async_copy(zg_hbm, zg_v, m2)

    def z(i, carry):
        hist[pl.ds(i * 16, 16)] = jnp.zeros((16,), jnp.float32)
        return carry

    lax.fori_loop(0, (npad * 2) // 16, z, 0)
    d0.wait()
    d1.wait()
    d2.wait()

    def b(i, carry):
        si = src_v[pl.ds(i * 16, 16)] * 2
        di = dst_v[pl.ds(i * 16, 16)] * 2
        v0 = plsc.load_gather(zg_v, [si])
        v1 = plsc.load_gather(zg_v, [si + 1])
        plsc.addupdate_scatter(hist, [di], v0)
        plsc.addupdate_scatter(hist, [di + 1], v1)
        return carry

    lax.fori_loop(0, eper // 16, b, 0)
    _publish_reduce(s, c, hist, shared, tmp, red, out_hbm,
                    (npad * 2) // _NS)


@functools.lru_cache(maxsize=None)
def _make_l2(npad, eper):
    seg = (npad * 2) // _NS
    return pl.kernel(
        functools.partial(_l2_body, eper, npad),
        out_type=jax.ShapeDtypeStruct((_NC, npad * 2), jnp.float32),
        mesh=plsc.VectorSubcoreMesh(**_MESH),
        compiler_params=_SC_PARAMS,
        scratch_types=[
            pltpu.VMEM((eper,), jnp.int32),
            pltpu.VMEM((eper,), jnp.int32),
            pltpu.VMEM((npad * 2,), jnp.float32),
            pltpu.VMEM((npad * 2,), jnp.float32),
            pltpu.VMEM((_NS, seg), jnp.float32),
            pltpu.VMEM((seg,), jnp.float32),
            pltpu.VMEM_SHARED((_NS, npad * 2), jnp.float32),
            pltpu.SemaphoreType.DMA,
            pltpu.SemaphoreType.DMA,
            pltpu.SemaphoreType.DMA,
        ],
    )


def _mm_body(x_ref, w_ref, o_ref):
    o_ref[...] = jnp.dot(x_ref[...], w_ref[...],
                         preferred_element_type=jnp.float32)


def _matmul(x, w, blk):
    m, k = x.shape
    _, n = w.shape
    return pl.pallas_call(
        _mm_body,
        grid=(m // blk,),
        in_specs=[
            pl.BlockSpec((blk, k), lambda i: (i, 0)),
            pl.BlockSpec((k, n), lambda i: (0, 0)),
        ],
        out_specs=pl.BlockSpec((blk, n), lambda i: (i, 0)),
        out_shape=jax.ShapeDtypeStruct((m, n), jnp.float32),
    )(x, w)


def _scale_body(d_ref, y_ref, o_ref):
    o_ref[...] = lax.rsqrt(d_ref[...]) * y_ref[...]


def _scale(degc, y, blk):
    m, k = y.shape
    return pl.pallas_call(
        _scale_body,
        grid=(m // blk,),
        in_specs=[
            pl.BlockSpec((blk, 1), lambda i: (i, 0)),
            pl.BlockSpec((blk, k), lambda i: (i, 0)),
        ],
        out_specs=pl.BlockSpec((blk, k), lambda i: (i, 0)),
        out_shape=jax.ShapeDtypeStruct((m, k), jnp.float32),
    )(degc, y)


def _combine_mm_body(d_ref, a_ref, z_ref, b_ref, w_ref, o_ref):
    dis = lax.rsqrt(d_ref[...])
    h = dis * (a_ref[0] + a_ref[1] - z_ref[...]) + b_ref[...]
    h = jnp.maximum(h, 0.0)
    o_ref[...] = dis * jnp.dot(h, w_ref[...],
                               preferred_element_type=jnp.float32)


def _combine_matmul(degc, acc, z, b1, w2, blk):
    m, k = z.shape
    n = w2.shape[1]
    return pl.pallas_call(
        _combine_mm_body,
        grid=(m // blk,),
        in_specs=[
            pl.BlockSpec((blk, 1), lambda i: (i, 0)),
            pl.BlockSpec((2, blk, k), lambda i: (0, i, 0)),
            pl.BlockSpec((blk, k), lambda i: (i, 0)),
            pl.BlockSpec((1, k), lambda i: (0, 0)),
            pl.BlockSpec((k, n), lambda i: (0, 0)),
        ],
        out_specs=pl.BlockSpec((blk, n), lambda i: (i, 0)),
        out_shape=jax.ShapeDtypeStruct((m, n), jnp.float32),
    )(degc, acc, z, b1, w2)


def _epilogue_body(d_ref, a2_ref, zg_ref, b2_ref, o_ref):
    dis = lax.rsqrt(d_ref[...])
    o_ref[...] = dis * (a2_ref[0] + a2_ref[1] + zg_ref[...]) + b2_ref[...]


def _epilogue(degc, acc2, zg2, b2r, n):
    return pl.pallas_call(
        _epilogue_body,
        grid=(1,),
        in_specs=[
            pl.BlockSpec((n, 1), lambda i: (0, 0)),
            pl.BlockSpec((2, n, 2), lambda i: (0, 0, 0)),
            pl.BlockSpec((n, 2), lambda i: (0, 0)),
            pl.BlockSpec((1, 2), lambda i: (0, 0)),
        ],
        out_specs=pl.BlockSpec((n, 2), lambda i: (0, 0)),
        out_shape=jax.ShapeDtypeStruct((n, 2), jnp.float32),
    )(degc, acc2, zg2, b2r)


def kernel(x, edge_index, W1, b1, W2, b2):
    n = x.shape[0]
    e = edge_index.shape[1]
    npad = ((n + 2047) // 2048) * 2048
    blk = npad // 5
    ei = edge_index.astype(jnp.int32)

    grain = _NW * 128 * 3
    epad = ((e + grain - 1) // grain) * grain
    eper = epad // _NW
    fill = n + (jnp.arange(epad - e, dtype=jnp.int32) % (npad - n))
    srcp = jnp.concatenate([ei[0], fill])
    dstp = jnp.concatenate([ei[1], fill])

    dacc = _make_deg(npad, eper)(dstp)
    xp = jnp.pad(x, ((0, npad - n), (0, 0)))
    y = _matmul(xp, W1, blk)
    degc = (dacc[0] + dacc[1] + 1.0)[:, None]

    z = _scale(degc, y, blk)
    acc = _make_spmm(npad, eper, 48)(z, srcp, dstp)
    zg2 = _combine_matmul(degc, acc, z, b1[None, :], W2, blk)

    s2 = _make_l2(npad, eper)(zg2.reshape(-1), srcp, dstp)
    return _epilogue(degc, s2.reshape(_NC, npad, 2), zg2,
                     b2[None, :], n)
